# baseline (speedup 1.0000x reference)
import functools

import jax
import jax.numpy as jnp
from jax import lax
from jax.experimental import pallas as pl
from jax.experimental.pallas import tpu as pltpu
from jax.experimental.pallas import tpu_sc as plsc

_POSE_NUM = 1000000
_EMBED_DIM = 6
_BATCH = 16384
_LANES = 128

_NC = 2
_NS = 16
_NW = _NC * _NS
_B_PER_W = _BATCH // _NW
_BB = 64
_NBATCH = _B_PER_W // _BB
_VEC = 16

_mesh = plsc.VectorSubcoreMesh(core_axis_name="c", subcore_axis_name="s")


@functools.partial(
    pl.kernel,
    mesh=_mesh,
    out_type=jax.ShapeDtypeStruct((_EMBED_DIM, _BATCH), jnp.float32),
    scratch_types=[
        pltpu.VMEM((_B_PER_W,), jnp.int32),
        pltpu.VMEM((_BB * 8, _LANES), jnp.float32),
        pltpu.VMEM((_EMBED_DIM, _B_PER_W), jnp.float32),
        pltpu.SemaphoreType.DMA,
    ],
    compiler_params=pltpu.CompilerParams(
        use_tc_tiling_on_sc=True, needs_layout_passes=False
    ),
)
def _gather_kernel(idx_hbm, table_hbm, out_hbm, idx_v, wins_v, rows_v, sem):
    wid = lax.axis_index("s") * _NC + lax.axis_index("c")
    base = wid * _B_PER_W
    pltpu.sync_copy(idx_hbm.at[pl.ds(base, _B_PER_W)], idx_v)
    lanes_iota = lax.iota(jnp.int32, _VEC)
    for b in range(_NBATCH):
        vecs = [
            idx_v[pl.ds(b * _BB + g * _VEC, _VEC)]
            for g in range(_BB // _VEC)
        ]
        copies = []
        for g, vec in enumerate(vecs):
            for k in range(_VEC):
                i = lax.index_in_dim(vec, k, axis=0, keepdims=False)
                blk = pl.multiple_of((i >> 7) << 7, _LANES)
                copies.append(
                    pltpu.async_copy(
                        table_hbm.at[pl.ds(0, _EMBED_DIM), pl.ds(blk, _LANES)],
                        wins_v.at[pl.ds((g * _VEC + k) * 8, _EMBED_DIM)],
                        sem,
                    )
                )
        for cp in copies:
            cp.wait()
        for g, vec in enumerate(vecs):
            slot = (lanes_iota + g * _VEC) * 8
            lane = lax.bitwise_and(vec, _LANES - 1)
            for c in range(_EMBED_DIM):
                vals = plsc.load_gather(wins_v, [slot + c, lane])
                rows_v[c, pl.ds(b * _BB + g * _VEC, _VEC)] = vals
    pltpu.sync_copy(
        rows_v, out_hbm.at[pl.ds(0, _EMBED_DIM), pl.ds(base, _B_PER_W)]
    )


def kernel(indices, table):
    table_t = table.T
    out_t = _gather_kernel(indices.astype(jnp.int32), table_t)
    return out_t.T


# speedup vs baseline: 1.0696x; 1.0696x over previous
import functools

import jax
import jax.numpy as jnp
from jax import lax
from jax.experimental import pallas as pl
from jax.experimental.pallas import tpu as pltpu
from jax.experimental.pallas import tpu_sc as plsc

_POSE_NUM = 1000000
_EMBED_DIM = 6
_BATCH = 16384
_LANES = 128

_NC = 2
_NS = 16
_NW = _NC * _NS
_B_PER_W = _BATCH // _NW
_BB = 32
_NBATCH = _B_PER_W // _BB
_VEC = 16

_mesh = plsc.VectorSubcoreMesh(core_axis_name="c", subcore_axis_name="s")


@functools.partial(
    pl.kernel,
    mesh=_mesh,
    out_type=jax.ShapeDtypeStruct((_EMBED_DIM, _BATCH), jnp.float32),
    scratch_types=[
        pltpu.VMEM((_B_PER_W,), jnp.int32),
        pltpu.VMEM((_BB * 8, _LANES), jnp.float32),
        pltpu.VMEM((_BB * 8, _LANES), jnp.float32),
        pltpu.VMEM((_EMBED_DIM, _B_PER_W), jnp.float32),
        pltpu.SemaphoreType.DMA,
        pltpu.SemaphoreType.DMA,
        pltpu.SemaphoreType.DMA,
    ],
    compiler_params=pltpu.CompilerParams(
        use_tc_tiling_on_sc=True, needs_layout_passes=False
    ),
)
def _gather_kernel(
    idx_hbm, table_hbm, out_hbm, idx_v, wins_a, wins_b, rows_v, sem_a, sem_b, sem_i
):
    wid = lax.axis_index("s") * _NC + lax.axis_index("c")
    base = wid * _B_PER_W
    pltpu.async_copy(idx_hbm.at[pl.ds(base, _B_PER_W)], idx_v, sem_i).wait()
    lanes_iota = lax.iota(jnp.int32, _VEC)
    slabs = (wins_a, wins_b)
    sems = (sem_a, sem_b)

    def fire(b):
        slab, sem = slabs[b % 2], sems[b % 2]
        vecs = [
            idx_v[pl.ds(b * _BB + g * _VEC, _VEC)]
            for g in range(_BB // _VEC)
        ]
        copies = []
        for g, vec in enumerate(vecs):
            for k in range(_VEC):
                i = lax.index_in_dim(vec, k, axis=0, keepdims=False)
                blk = pl.multiple_of((i >> 7) << 7, _LANES)
                copies.append(
                    pltpu.async_copy(
                        table_hbm.at[pl.ds(0, _EMBED_DIM), pl.ds(blk, _LANES)],
                        slab.at[pl.ds((g * _VEC + k) * 8, _EMBED_DIM)],
                        sem,
                    )
                )
        return vecs, copies

    def extract(b, vecs, copies):
        slab = slabs[b % 2]
        for cp in copies:
            cp.wait()
        for g, vec in enumerate(vecs):
            slot = (lanes_iota + g * _VEC) * 8
            lane = lax.bitwise_and(vec, _LANES - 1)
            for c in range(_EMBED_DIM):
                vals = plsc.load_gather(slab, [slot + c, lane])
                rows_v[c, pl.ds(b * _BB + g * _VEC, _VEC)] = vals

    prev = fire(0)
    for b in range(1, _NBATCH):
        cur = fire(b)
        extract(b - 1, *prev)
        prev = cur
    extract(_NBATCH - 1, *prev)
    pltpu.sync_copy(
        rows_v, out_hbm.at[pl.ds(0, _EMBED_DIM), pl.ds(base, _B_PER_W)]
    )


def kernel(indices, table):
    table_t = table.T
    out_t = _gather_kernel(indices.astype(jnp.int32), table_t)
    return out_t.T


# speedup vs baseline: 1.0748x; 1.0049x over previous
import functools

import jax
import jax.numpy as jnp
from jax import lax
from jax.experimental import pallas as pl
from jax.experimental.pallas import tpu as pltpu
from jax.experimental.pallas import tpu_sc as plsc

_POSE_NUM = 1000000
_EMBED_DIM = 6
_BATCH = 16384
_LANES = 128

_NC = 2
_NS = 16
_NW = _NC * _NS
_B_PER_W = _BATCH // _NW
_BB = 32
_NBATCH = _B_PER_W // _BB
_VEC = 16

_mesh = plsc.VectorSubcoreMesh(core_axis_name="c", subcore_axis_name="s")


@functools.partial(
    pl.kernel,
    mesh=_mesh,
    out_type=jax.ShapeDtypeStruct((_EMBED_DIM, _BATCH), jnp.float32),
    scratch_types=[
        pltpu.VMEM((_B_PER_W,), jnp.int32),
        pltpu.VMEM((_BB * 8, _LANES), jnp.float32),
        pltpu.VMEM((_BB * 8, _LANES), jnp.float32),
        pltpu.VMEM((_EMBED_DIM, _B_PER_W), jnp.float32),
        pltpu.SemaphoreType.DMA,
        pltpu.SemaphoreType.DMA,
        pltpu.SemaphoreType.DMA,
    ],
    compiler_params=pltpu.CompilerParams(
        use_tc_tiling_on_sc=True,
        needs_layout_passes=False,
        skip_device_barrier=True,
    ),
)
def _gather_kernel(
    idx_hbm, table_hbm, out_hbm, idx_v, wins_a, wins_b, rows_v, sem_a, sem_b, sem_i
):
    wid = lax.axis_index("s") * _NC + lax.axis_index("c")
    base = wid * _B_PER_W
    pltpu.async_copy(idx_hbm.at[pl.ds(base, _B_PER_W)], idx_v, sem_i).wait()
    lanes_iota = lax.iota(jnp.int32, _VEC)
    slabs = (wins_a, wins_b)
    sems = (sem_a, sem_b)

    def fire(b):
        slab, sem = slabs[b % 2], sems[b % 2]
        vecs = [
            idx_v[pl.ds(b * _BB + g * _VEC, _VEC)]
            for g in range(_BB // _VEC)
        ]
        copies = []
        for g, vec in enumerate(vecs):
            for k in range(_VEC):
                i = lax.index_in_dim(vec, k, axis=0, keepdims=False)
                blk = pl.multiple_of((i >> 7) << 7, _LANES)
                copies.append(
                    pltpu.async_copy(
                        table_hbm.at[pl.ds(0, _EMBED_DIM), pl.ds(blk, _LANES)],
                        slab.at[pl.ds((g * _VEC + k) * 8, _EMBED_DIM)],
                        sem,
                    )
                )
        return vecs, copies

    def extract(b, vecs, copies):
        slab = slabs[b % 2]
        for cp in copies:
            cp.wait()
        for g, vec in enumerate(vecs):
            slot = (lanes_iota + g * _VEC) * 8
            lane = lax.bitwise_and(vec, _LANES - 1)
            for c in range(_EMBED_DIM):
                vals = plsc.load_gather(slab, [slot + c, lane])
                rows_v[c, pl.ds(b * _BB + g * _VEC, _VEC)] = vals

    prev = fire(0)
    for b in range(1, _NBATCH):
        cur = fire(b)
        extract(b - 1, *prev)
        prev = cur
    extract(_NBATCH - 1, *prev)
    pltpu.sync_copy(
        rows_v, out_hbm.at[pl.ds(0, _EMBED_DIM), pl.ds(base, _B_PER_W)]
    )


def kernel(indices, table):
    table_t = table.T
    out_t = _gather_kernel(indices.astype(jnp.int32), table_t)
    return out_t.T
